# baseline (speedup 1.0000x reference)
import functools

import jax
import jax.numpy as jnp
from jax import lax
from jax.experimental import pallas as pl
from jax.experimental.pallas import tpu as pltpu
from jax.experimental.pallas import tpu_sc as plsc

_N = 10000

_P1A_ROWS = 5120
_P1A_BLK = 512
_P1B_ROWS = _N - _P1A_ROWS
_P1B_BLK = 488

_BLK2 = 400
_NB2 = 11

_SC_R = 5888
_S0 = _N - _SC_R
_RPT = _SC_R // 32
_NGRP = _RPT // 8

_CHUNKS_A = ((0, 0, 2560), (2560, 2560, 2560))
_CHUNKS_B = ((5120, 0, 2560), (7680, 2560, 2304), (9984, 4864, 16))
_WG_A = _P1A_ROWS
_WG_B = _P1B_ROWS


def _pass1_body(nblk, row0, need_u, x_ref, W1_ref, b1_ref, W2_ref, adj_ref,
                u_in_ref, gt_ref, u_out_ref, u_ref, h_ref):
    i = pl.program_id(0)
    blk = adj_ref.shape[0]

    @pl.when(i == 0)
    def _():
        if need_u:
            u_ref[...] = jnp.dot(x_ref[...], W1_ref[...],
                                 preferred_element_type=jnp.float32)
        else:
            u_ref[...] = u_in_ref[...]

    acc = jnp.dot(adj_ref[...], u_ref[...], preferred_element_type=jnp.float32)
    h_ref[pl.ds(i * blk, blk), :] = jnp.maximum(acc + b1_ref[...], 0.0)

    @pl.when(i == nblk - 1)
    def _():
        gt_ref[...] = lax.dot_general(
            W2_ref[...], h_ref[...], (((0,), (1,)), ((), ())),
            preferred_element_type=jnp.float32)
        if need_u:
            u_out_ref[...] = u_ref[...]


def _pass1(x, W1, b1r, W2, adj, row0, nrows, blk, u_in):
    need_u = u_in is None
    nblk = nrows // blk
    b0 = row0 // blk
    out_shapes = [jax.ShapeDtypeStruct((2, nrows), jnp.float32)]
    out_specs = [pl.BlockSpec((2, nrows), lambda i: (0, 0))]
    inner = functools.partial(_pass1_body, nblk, row0, need_u)
    in_specs = [
        pl.BlockSpec((_N, 128), lambda i: (0, 0)),
        pl.BlockSpec((128, 16), lambda i: (0, 0)),
        pl.BlockSpec((1, 16), lambda i: (0, 0)),
        pl.BlockSpec((16, 2), lambda i: (0, 0)),
        pl.BlockSpec((blk, _N), lambda i: (b0 + i, 0)),
    ]
    args = [x, W1, b1r, W2, adj]
    if need_u:
        out_shapes.append(jax.ShapeDtypeStruct((_N, 16), jnp.float32))
        out_specs.append(pl.BlockSpec((_N, 16), lambda i: (0, 0)))

        def body(x_ref, W1_ref, b1_ref, W2_ref, adj_ref, gt_ref, u_out_ref,
                 u_ref, h_ref):
            inner(x_ref, W1_ref, b1_ref, W2_ref, adj_ref, None, gt_ref,
                  u_out_ref, u_ref, h_ref)
    else:
        in_specs.append(pl.BlockSpec((_N, 16), lambda i: (0, 0)))
        args.append(u_in)

        def body(x_ref, W1_ref, b1_ref, W2_ref, adj_ref, u_in_ref, gt_ref,
                 u_ref, h_ref):
            inner(x_ref, W1_ref, b1_ref, W2_ref, adj_ref, u_in_ref,
                  gt_ref, None, u_ref, h_ref)

    res = pl.pallas_call(
        body,
        grid=(nblk,),
        in_specs=in_specs,
        out_specs=out_specs,
        out_shape=out_shapes,
        scratch_shapes=[
            pltpu.VMEM((_N, 16), jnp.float32),
            pltpu.VMEM((nrows, 16), jnp.float32),
        ],
    )(*args)
    if need_u:
        return res[0], res[1]
    return res[0], None


def _pass2_body(ga_ref, gb_ref, adj_ref, m_ref, acc_ref):
    i = pl.program_id(0)

    @pl.when(i == 0)
    def _():
        acc_ref[...] = jnp.full_like(acc_ref, -jnp.inf)

    part = lax.dot_general(adj_ref[:, : _P1A_ROWS], ga_ref[...],
                           (((1,), (1,)), ((), ())),
                           preferred_element_type=jnp.float32)
    part = part + lax.dot_general(adj_ref[:, _P1A_ROWS:], gb_ref[...],
                                  (((1,), (1,)), ((), ())),
                                  preferred_element_type=jnp.float32)
    acc_ref[...] = jnp.maximum(acc_ref[...],
                               jnp.max(part, axis=0, keepdims=True))

    @pl.when(i == _NB2 - 1)
    def _():
        m_ref[...] = acc_ref[...]


def _sc_core(chunks, wg, adj_hbm, g_hbm, out_hbm, g_buf, bufs, stage, sems):
    nchunk = len(chunks)
    wid = lax.axis_index("s") * 2 + lax.axis_index("c")
    row0 = _S0 + wid * _RPT

    pltpu.sync_copy(g_hbm, g_buf)

    def copy_obj(grp, cc):
        return pltpu.make_async_copy(
            adj_hbm.at[pl.ds(row0 + grp * 8, 8),
                       pl.ds(chunks[cc][0], chunks[cc][2])],
            bufs[cc], sems[cc])

    for cc in range(nchunk):
        copy_obj(0, cc).start()

    zero = jnp.zeros((16,), jnp.float32)

    def grp_body(grp, _):
        accs = [zero] * 16
        for cc in range(nchunk):
            copy_obj(grp, cc).wait()
            buf = bufs[cc]
            gl0 = chunks[cc][1]
            clen = chunks[cc][2]

            if clen > 16:
                @plsc.parallel_loop(0, clen, 16, unroll=2,
                                    carry=tuple(accs))
                def accs_new(p, carry):
                    g0 = g_buf[0, pl.ds(gl0 + p, 16)]
                    g1 = g_buf[1, pl.ds(gl0 + p, 16)]
                    out = []
                    for r in range(8):
                        a = buf[r, pl.ds(p, 16)]
                        out.append(carry[2 * r] + a * g0)
                        out.append(carry[2 * r + 1] + a * g1)
                    return tuple(out)

                accs = list(accs_new)
            else:
                g0 = g_buf[0, pl.ds(gl0, 16)]
                g1 = g_buf[1, pl.ds(gl0, 16)]
                new = []
                for r in range(8):
                    a = buf[r, :]
                    new.append(accs[2 * r] + a * g0)
                    new.append(accs[2 * r + 1] + a * g1)
                accs = new

            @pl.when(grp + 1 < _NGRP)
            def _():
                copy_obj(grp + 1, cc).start()

        for r in range(8):
            for c in range(2):
                stage[r, pl.ds(c * 16, 16)] = accs[2 * r + c]
        pltpu.sync_copy(
            stage, out_hbm.at[pl.ds(wid * _RPT + grp * 8, 8), :])
        return 0

    lax.fori_loop(0, _NGRP, grp_body, 0)


def _sc_layer2(adj, g_flat, chunks, wg):
    nchunk = len(chunks)

    def body(adj_hbm, g_hbm, out_hbm, *scratch):
        g_buf = scratch[0]
        bufs = scratch[1:1 + nchunk]
        stage = scratch[1 + nchunk]
        sems = scratch[2 + nchunk:]
        _sc_core(chunks, wg, adj_hbm, g_hbm, out_hbm, g_buf, bufs, stage,
                 sems)

    mesh = plsc.VectorSubcoreMesh(core_axis_name="c", subcore_axis_name="s")
    scratch_types = [pltpu.VMEM((2, wg), jnp.float32)]
    scratch_types += [pltpu.VMEM((8, c[2]), jnp.float32) for c in chunks]
    scratch_types += [pltpu.VMEM((8, 32), jnp.float32)]
    scratch_types += [pltpu.SemaphoreType.DMA for _ in chunks]
    sc = pl.kernel(
        body,
        out_type=jax.ShapeDtypeStruct((_SC_R, 32), jnp.float32),
        mesh=mesh,
        scratch_types=scratch_types,
    )
    return sc(adj, g_flat)


def _reduce_body(s_ref, out_ref):
    a = s_ref[...]
    s0 = jnp.sum(a[:, 0:16], axis=1)
    s1 = jnp.sum(a[:, 16:32], axis=1)
    out_ref[...] = jnp.concatenate([s0[:, None], s1[:, None]], axis=1)


def _reduce_lanes(sums):
    return pl.pallas_call(
        _reduce_body,
        in_specs=[pl.BlockSpec((_SC_R, 32), lambda: (0, 0))],
        out_specs=pl.BlockSpec((_SC_R, 2), lambda: (0, 0)),
        out_shape=jax.ShapeDtypeStruct((_SC_R, 2), jnp.float32),
    )(sums)


def _combine_body(ra_ref, rb_ref, mtc_ref, b2_ref, W3_ref, b3_ref, out_ref):
    a = ra_ref[...] + rb_ref[...]
    m0 = jnp.maximum(jnp.max(a[:, 0]), mtc_ref[0, 0]) + b2_ref[0, 0]
    m1 = jnp.maximum(jnp.max(a[:, 1]), mtc_ref[0, 1]) + b2_ref[0, 1]
    val = m0 * W3_ref[0, 0] + m1 * W3_ref[1, 0] + b3_ref[0, 0]
    out_ref[...] = jnp.full((1, 1), val, dtype=jnp.float32)


@jax.jit
def kernel(x, adj, W1, b1, W2, b2, W3, b3):
    b1r = b1.reshape(1, 16)
    b2r = b2.reshape(1, 2)
    b3r = b3.reshape(1, 1)

    g_a, u = _pass1(x, W1, b1r, W2, adj, 0, _P1A_ROWS, _P1A_BLK, None)
    sums_a = _sc_layer2(adj, g_a, _CHUNKS_A, _WG_A)

    g_b, _ = _pass1(x, W1, b1r, W2, adj, _P1A_ROWS, _P1B_ROWS, _P1B_BLK, u)
    ra = _reduce_lanes(sums_a)
    sums_b = _sc_layer2(adj, g_b, _CHUNKS_B, _WG_B)

    m_tc = pl.pallas_call(
        _pass2_body,
        grid=(_NB2,),
        in_specs=[
            pl.BlockSpec((2, _P1A_ROWS), lambda i: (0, 0)),
            pl.BlockSpec((2, _P1B_ROWS), lambda i: (0, 0)),
            pl.BlockSpec((_BLK2, _N), lambda i: (i, 0)),
        ],
        out_specs=pl.BlockSpec((1, 2), lambda i: (0, 0)),
        out_shape=jax.ShapeDtypeStruct((1, 2), jnp.float32),
        scratch_shapes=[pltpu.VMEM((1, 2), jnp.float32)],
    )(g_a, g_b, adj)

    rb = _reduce_lanes(sums_b)

    out = pl.pallas_call(
        _combine_body,
        in_specs=[
            pl.BlockSpec((_SC_R, 2), lambda: (0, 0)),
            pl.BlockSpec((_SC_R, 2), lambda: (0, 0)),
            pl.BlockSpec((1, 2), lambda: (0, 0)),
            pl.BlockSpec((1, 2), lambda: (0, 0)),
            pl.BlockSpec((2, 1), lambda: (0, 0)),
            pl.BlockSpec((1, 1), lambda: (0, 0)),
        ],
        out_specs=pl.BlockSpec((1, 1), lambda: (0, 0)),
        out_shape=jax.ShapeDtypeStruct((1, 1), jnp.float32),
    )(ra, rb, m_tc, b2r, W3, b3r)

    return out.reshape(1, 1, 1)


# speedup vs baseline: 1.3276x; 1.3276x over previous
import jax
import jax.numpy as jnp
from jax.experimental import pallas as pl
from jax.experimental.pallas import tpu as pltpu

_N = 10000
_BLK = 400
_NB = _N // _BLK


def _body(x_ref, W1_ref, b1_ref, W2_ref, b2_ref, W3_ref, b3_ref, adj_ref,
          out_ref, u_ref, h_ref, g_ref, m_ref):
    i = pl.program_id(0)

    @pl.when(i == 0)
    def _():
        u_ref[...] = jnp.dot(x_ref[...], W1_ref[...],
                             preferred_element_type=jnp.float32)

    @pl.when(i < _NB)
    def _():
        acc = jnp.dot(adj_ref[...], u_ref[...],
                      preferred_element_type=jnp.float32)
        h_ref[pl.ds(i * _BLK, _BLK), :] = jnp.maximum(acc + b1_ref[...], 0.0)

    @pl.when(i == _NB)
    def _():
        g_ref[...] = jnp.dot(h_ref[...], W2_ref[...],
                             preferred_element_type=jnp.float32)
        m_ref[...] = jnp.full_like(m_ref, -jnp.inf)

    @pl.when(i >= _NB)
    def _():
        part = jnp.dot(adj_ref[...], g_ref[...],
                       preferred_element_type=jnp.float32) + b2_ref[...]
        m_ref[...] = jnp.maximum(m_ref[...],
                                 jnp.max(part, axis=0, keepdims=True))

    @pl.when(i == 2 * _NB - 1)
    def _():
        out_ref[...] = jnp.dot(m_ref[...], W3_ref[...],
                               preferred_element_type=jnp.float32) + b3_ref[...]


@jax.jit
def kernel(x, adj, W1, b1, W2, b2, W3, b3):
    n, nfeat = x.shape
    nhid = W1.shape[1]
    nout = W2.shape[1]

    b1r = b1.reshape(1, nhid)
    b2r = b2.reshape(1, nout)
    b3r = b3.reshape(1, 1)

    def adj_idx(i):
        return (jnp.where(i < _NB, i, 2 * _NB - 1 - i), 0)

    out = pl.pallas_call(
        _body,
        grid=(2 * _NB,),
        in_specs=[
            pl.BlockSpec((n, nfeat), lambda i: (0, 0)),
            pl.BlockSpec((nfeat, nhid), lambda i: (0, 0)),
            pl.BlockSpec((1, nhid), lambda i: (0, 0)),
            pl.BlockSpec((nhid, nout), lambda i: (0, 0)),
            pl.BlockSpec((1, nout), lambda i: (0, 0)),
            pl.BlockSpec((nout, 1), lambda i: (0, 0)),
            pl.BlockSpec((1, 1), lambda i: (0, 0)),
            pl.BlockSpec((_BLK, n), adj_idx),
        ],
        out_specs=pl.BlockSpec((1, 1), lambda i: (0, 0)),
        out_shape=jax.ShapeDtypeStruct((1, 1), jnp.float32),
        scratch_shapes=[
            pltpu.VMEM((n, nhid), jnp.float32),
            pltpu.VMEM((n, nhid), jnp.float32),
            pltpu.VMEM((n, nout), jnp.float32),
            pltpu.VMEM((1, nout), jnp.float32),
        ],
    )(x, W1, b1r, W2, b2r, W3, b3r, adj)

    return out.reshape(1, 1, 1)
